# Initial kernel scaffold; baseline (speedup 1.0000x reference)
import jax
import jax.numpy as jnp
from jax.experimental import pallas as pl


def kernel(x, edge_index, batch, Wrel1, brel1, Wroot1, Wrel2, brel2, Wroot2, Wrel3, brel3, Wroot3, Wrel4, brel4, Wroot4, pool_attn, lin1_W, lin1_b, lin2_W, lin2_b):
    raise NotImplementedError("write your pallas kernel here")


# speedup vs baseline: 4.6745x; 4.6745x over previous
import functools

import jax
import jax.numpy as jnp
from jax import lax
from jax.experimental import pallas as pl
from jax.experimental.pallas import tpu as pltpu
from jax.experimental.pallas import tpu_sc as plsc

_N = 10000
_E = 320000
_F = 128
_H = 256
_G = 64
_C = 10
_RATIO = 0.8

_NP = 10240
_BLK = 1024
_NB = _NP // _BLK


_NSUB = 16
_EPT = _E // _NSUB
_CH = 80
_NCH = _EPT // _CH
_SLAB = _NP // _NSUB


_HNP = _NP // 2
_ACC = _HNP + 8
_QS = _HNP // _NSUB
_LB = 25
_NRF = _NCH // _LB
_BW = 384


def _agg_body(table_ref, idx_ref, dsta_ref, bits_ref, zeros_ref, zrow_ref,
              agg_out_ref, deg_out_ref, idx_v, dsta_v, rows_a, rows_b,
              bits_v, dacc_v, acc, sem_a, sem_b, sem_c, sem_d):
    c = lax.axis_index("c")
    s = lax.axis_index("s")
    half = _NCH // 2

    pltpu.sync_copy(zeros_ref, acc.at[pl.ds(s * _QS, _QS)])
    pltpu.sync_copy(bits_ref, bits_v)
    pltpu.sync_copy(zrow_ref, dacc_v)
    plsc.subcore_barrier()

    def buf(j2):
        return (rows_a, sem_a) if j2 % 2 == 0 else (rows_b, sem_b)

    def ssem(j2):
        return sem_c if j2 % 2 == 0 else sem_d

    def s_wait(j2):
        bj, _ = buf(j2)
        pltpu.make_async_copy(bj, acc.at[dsta_v.at[j2]], ssem(j2)).wait()

    fv = plsc.load_gather(bits_v, [jnp.full((16,), _BW - 1, jnp.int32)])
    has_flag = lax.reduce_max(fv, (0,)) != 0
    do_agg = jnp.logical_or(has_flag, c == 0)

    def refill(r, carry):
        pltpu.sync_copy(idx_ref.at[c, s, r], idx_v)
        pltpu.sync_copy(dsta_ref.at[s, r], dsta_v)

        @pl.when(do_agg)
        def _():
            b0, s0 = buf(0)
            pltpu.async_copy(table_ref.at[idx_v.at[0]], b0, s0)

        for j2 in range(_LB):

            @pl.when(do_agg)
            def _(j2=j2):
                bj, sj = buf(j2)
                pltpu.make_async_copy(table_ref.at[idx_v.at[j2]], bj,
                                      sj).wait()
                if j2 + 1 < _LB:
                    if j2 >= 1:
                        s_wait(j2 - 1)
                    bn, sn = buf(j2 + 1)
                    pltpu.async_copy(table_ref.at[idx_v.at[j2 + 1]], bn, sn)
                pltpu.async_copy(bj, acc.at[dsta_v.at[j2]], ssem(j2),
                                 add=True)

            j = r * _LB + j2
            do_deg = lax.select(
                c == 0,
                jnp.logical_or(jnp.logical_not(has_flag), j < half),
                jnp.logical_and(has_flag, j >= half))

            @pl.when(do_deg)
            def _(j2=j2):
                for k in range(_CH // 16):
                    iv = idx_v[j2, pl.ds(k * 16, 16)]
                    sv = jnp.right_shift(iv, 1)
                    dv = dsta_v[j2, pl.ds(k * 16, 16)]
                    w = plsc.load_gather(bits_v, [jnp.right_shift(sv, 5)])
                    kv = jnp.bitwise_and(
                        jnp.right_shift(w, jnp.bitwise_and(sv, 31)),
                        1).astype(jnp.float32)
                    plsc.addupdate_scatter(dacc_v, [dv], kv,
                                           mask=dv < _HNP)

        @pl.when(do_agg)
        def _():
            s_wait(_LB - 2)
            s_wait(_LB - 1)

        return carry

    lax.fori_loop(0, _NRF, refill, 0)

    plsc.subcore_barrier()
    pltpu.sync_copy(acc.at[pl.ds(s * _QS, _QS)],
                    agg_out_ref.at[c, pl.ds(s * _QS, _QS)])
    pltpu.sync_copy(dacc_v, deg_out_ref.at[c, s])


def _agg_call(table, idx3, dsta3, bits):
    zeros = jnp.zeros((_QS, 128), jnp.float32)
    zrow = jnp.zeros((_HNP,), jnp.float32)
    mesh = plsc.VectorSubcoreMesh(core_axis_name="c", subcore_axis_name="s")
    out_type = [jax.ShapeDtypeStruct((2, _HNP, 128), jnp.float32),
                jax.ShapeDtypeStruct((2, _NSUB, _HNP), jnp.float32)]
    scratch = [
        pltpu.VMEM((_LB, _CH), jnp.int32),
        pltpu.VMEM((_LB, _CH), jnp.int32),
        pltpu.VMEM((_CH, 128), jnp.float32),
        pltpu.VMEM((_CH, 128), jnp.float32),
        pltpu.VMEM((_BW,), jnp.int32),
        pltpu.VMEM((_HNP,), jnp.float32),
        pltpu.VMEM_SHARED((_ACC, 128), jnp.float32),
        pltpu.SemaphoreType.DMA,
        pltpu.SemaphoreType.DMA,
        pltpu.SemaphoreType.DMA,
        pltpu.SemaphoreType.DMA,
    ]
    fn = pl.kernel(_agg_body, out_type=out_type, mesh=mesh,
                   scratch_types=scratch, name="edge_agg",
                   compiler_params=pltpu.CompilerParams(
                       needs_layout_passes=False))
    idx5 = idx3.reshape(2, _NSUB, _NRF, _LB, _CH)
    dsta5 = dsta3.reshape(_NSUB, _NRF, _LB, _CH)
    return fn(table, idx5, dsta5, bits.reshape(_BW), zeros, zrow)


def _conv_body(agg_ref, deg_ref, hin_ref, bat_ref, nm_ref, wr_ref, br_ref,
               ws_ref, pa_ref, h_ref, pool_ref, cnt_ref, score_ref,
               starts_ref):
    i = pl.program_id(0)
    d = deg_ref[:, :, pl.ds(i * _BLK, _BLK)]
    deg = jnp.sum(d, axis=(0, 1))
    hin = hin_ref[...]
    nm = nm_ref[0, pl.ds(i * _BLK, _BLK)]
    inv = (1.0 / jnp.maximum(deg, 1.0))[:, None]
    rel = None
    for cc in range(2):
        col = cc * 128
        t = lax.dot_general(agg_ref[cc] * inv, wr_ref[:, col:col + 128],
                            (((1,), (1,)), ((), ())),
                            preferred_element_type=jnp.float32)
        rel = t if rel is None else rel + t
    root = lax.dot_general(hin, ws_ref[...], (((1,), (1,)), ((), ())),
                           preferred_element_type=jnp.float32)
    h = jax.nn.relu(rel + br_ref[0, :][None, :] + root)
    h_ref[...] = h * nm[:, None]

    bat = bat_ref[0, pl.ds(i * _BLK, _BLK)]
    gids = lax.broadcasted_iota(jnp.int32, (_G, _BLK), 0)
    oh = jnp.where(gids == bat[None, :], nm[None, :], 0.0)
    psum = lax.dot_general(oh, h, (((1,), (0,)), ((), ())),
                           preferred_element_type=jnp.float32)
    pcnt = jnp.sum(oh, axis=1)

    @pl.when(i == 0)
    def _():
        pool_ref[...] = jnp.zeros_like(pool_ref)
        cnt_ref[...] = jnp.zeros_like(cnt_ref)

    pool_ref[...] += psum
    row0 = lax.broadcasted_iota(jnp.int32, (8, _G), 0) == 0
    cnt_ref[...] += jnp.where(row0, pcnt[None, :], 0.0)

    pa = pa_ref[0, :]
    nrm = jnp.sqrt(jnp.sum(pa * pa)) + 1e-12
    s = jnp.sum(h * (pa / nrm)[None, :], axis=1)
    score_ref[0, pl.ds(i * _BLK, _BLK)] = s

    @pl.when(i == _NB - 1)
    def _():
        counts = cnt_ref[0, :]
        gp = lax.broadcasted_iota(jnp.int32, (_G, 128), 0)
        ln = lax.broadcasted_iota(jnp.int32, (_G, 128), 1)
        lt = (gp < ln).astype(jnp.float32)
        starts_ref[...] = lax.dot_general(
            counts[None, :], lt, (((1,), (0,)), ((), ())),
            preferred_element_type=jnp.float32)


def _conv_call(agg, deg, hin, bat, nm, Wr, br, Ws, pa, *, with_score=True):
    fin = hin.shape[1]
    out_shape = [
        jax.ShapeDtypeStruct((_NP, _H), jnp.float32),
        jax.ShapeDtypeStruct((_G, _H), jnp.float32),
        jax.ShapeDtypeStruct((8, _G), jnp.float32),
        jax.ShapeDtypeStruct((1, _NP), jnp.float32),
        jax.ShapeDtypeStruct((1, 128), jnp.float32),
    ]
    full = lambda shape: pl.BlockSpec(shape, lambda i: (0, 0))
    return pl.pallas_call(
        _conv_body,
        grid=(_NB,),
        in_specs=[
            pl.BlockSpec((2, _BLK, 128), lambda i: (0, i, 0)),
            pl.BlockSpec((2, _NSUB, _NP), lambda i: (0, 0, 0)),
            pl.BlockSpec((_BLK, fin), lambda i: (i, 0)),
            full((1, _NP)),
            full((1, _NP)),
            full((_H, fin)),
            full((1, _H)),
            full((_H, fin)),
            full((1, _H)),
        ],
        out_specs=[
            pl.BlockSpec((_BLK, _H), lambda i: (i, 0)),
            full((_G, _H)),
            full((8, _G)),
            full((1, _NP)),
            full((1, 128)),
        ],
        out_shape=out_shape,
    )(agg, deg, hin, bat, nm, Wr, br, Ws, pa)


def _mask_body(h_ref, score_ref, keep_ref, out_ref):
    i = pl.program_id(0)
    s = score_ref[0, pl.ds(i * _BLK, _BLK)]
    k = keep_ref[0, pl.ds(i * _BLK, _BLK)]
    out_ref[...] = h_ref[...] * (jnp.tanh(s) * k)[:, None]


def _mask_call(h, score, keep):
    full = lambda shape: pl.BlockSpec(shape, lambda i: (0, 0))
    return pl.pallas_call(
        _mask_body,
        grid=(_NB,),
        in_specs=[
            pl.BlockSpec((_BLK, _H), lambda i: (i, 0)),
            full((1, _NP)),
            full((1, _NP)),
        ],
        out_specs=pl.BlockSpec((_BLK, _H), lambda i: (i, 0)),
        out_shape=jax.ShapeDtypeStruct((_NP, _H), jnp.float32),
    )(h, score, keep)


def _head_body(p1_ref, p2_ref, p3_ref, p4_ref, c0_ref, c1_ref, w1_ref,
               b1_ref, w2_ref, b2_ref, out_ref):
    ic0 = 1.0 / jnp.maximum(c0_ref[0, :], 1.0)
    ic1 = 1.0 / jnp.maximum(c1_ref[0, :], 1.0)
    z = jnp.concatenate([
        p1_ref[...] * ic0[:, None],
        p2_ref[...] * ic0[:, None],
        p3_ref[...] * ic1[:, None],
        p4_ref[...] * ic1[:, None],
    ], axis=1)
    t = lax.dot_general(z, w1_ref[...], (((1,), (1,)), ((), ())),
                        preferred_element_type=jnp.float32)
    t = jax.nn.relu(t + b1_ref[0, :][None, :])
    o = lax.dot_general(t, w2_ref[...], (((1,), (1,)), ((), ())),
                        preferred_element_type=jnp.float32)
    o = o + b2_ref[0, :][None, :]
    lane = lax.broadcasted_iota(jnp.int32, (_G, 128), 1)
    o = jnp.where(lane < _C, o, -jnp.inf)
    m = jnp.max(o, axis=1, keepdims=True)
    e = jnp.where(lane < _C, jnp.exp(o - m), 0.0)
    out_ref[...] = e / jnp.sum(e, axis=1, keepdims=True)


def _head_call(p1, p2, p3, p4, c0, c1, w1, b1, w2, b2):
    full = lambda shape: pl.BlockSpec(shape, lambda: tuple(0 for _ in shape))
    return pl.pallas_call(
        _head_body,
        in_specs=[
            full((_G, _H)), full((_G, _H)), full((_G, _H)), full((_G, _H)),
            full((8, _G)), full((8, _G)),
            full((_H, 4 * _H)), full((1, _H)),
            full((128, _H)), full((1, 128)),
        ],
        out_specs=full((_G, 128)),
        out_shape=jax.ShapeDtypeStruct((_G, 128), jnp.float32),
    )(p1, p2, p3, p4, c0, c1, w1, b1, w2, b2)


_TSL = _NP // 32


def _topk_body(score_ref, starts_in_ref, keep_out_ref, score_v, starts_f,
               starts_v, kcnt_v, keep_v):
    c = lax.axis_index("c")
    s = lax.axis_index("s")
    w = s * 2 + c
    base = w * _TSL
    pltpu.sync_copy(score_ref, score_v)
    pltpu.sync_copy(starts_in_ref, starts_f)

    for k in range(8):
        starts_v[pl.ds(16 * k, 16)] = (
            starts_f[pl.ds(16 * k, 16)].astype(jnp.int32))

    for k in range(8):
        gv = lax.iota(jnp.int32, 16) + 16 * k
        st0 = plsc.load_gather(starts_v, [gv])
        st1 = plsc.load_gather(starts_v, [jnp.minimum(gv + 1, 127)])
        cf = (st1 - st0).astype(jnp.float32)
        v = jnp.float32(_RATIO) * cf
        t = v.astype(jnp.int32)
        kc = t + (t.astype(jnp.float32) < v).astype(jnp.int32)
        kcnt_v[pl.ds(16 * k, 16)] = jnp.where(gv < _G, kc, 0)

    def chunk_body(k, carry):
        b16 = base + k * 16
        i_vec = lax.iota(jnp.int32, 16) + b16
        s_i = score_v[pl.ds(b16, 16)]
        lo = jnp.zeros((16,), jnp.int32)
        hi = jnp.full((16,), 127, jnp.int32)
        for _ in range(7):
            mid = (lo + hi + 1) // 2
            v = plsc.load_gather(starts_v, [mid])
            ok = v <= i_vec
            lo = jnp.where(ok, mid, lo)
            hi = jnp.where(ok, hi, mid - 1)
        b_i = lo
        jlo_v = plsc.load_gather(starts_v, [b_i])
        jhi_v = plsc.load_gather(starts_v, [jnp.minimum(b_i + 1, 127)])
        jmin = lax.reduce_min(jlo_v, (0,))
        jmax = lax.reduce_max(jhi_v, (0,))

        def inner(j, cnt):
            jb = jnp.zeros((16,), jnp.int32) + j
            sj = plsc.load_gather(score_v, [jb])
            gt = jnp.logical_or(sj > s_i,
                                jnp.logical_and(sj == s_i, j < i_vec))
            m = jnp.logical_and(gt, jnp.logical_and(j >= jlo_v, j < jhi_v))
            return cnt + m.astype(jnp.int32)

        cnt = lax.fori_loop(jmin, jmax, inner, jnp.zeros((16,), jnp.int32))
        kc = plsc.load_gather(kcnt_v, [b_i])
        keep_v[pl.ds(k * 16, 16)] = (cnt < kc).astype(jnp.float32)
        return carry

    lax.fori_loop(0, _TSL // 16, chunk_body, 0)
    pltpu.sync_copy(keep_v, keep_out_ref.at[pl.ds(base, _TSL)])


def _topk_call(score_flat, starts_row):
    mesh = plsc.VectorSubcoreMesh(core_axis_name="c", subcore_axis_name="s")
    scratch = [
        pltpu.VMEM((_NP,), jnp.float32),
        pltpu.VMEM((128,), jnp.float32),
        pltpu.VMEM((128,), jnp.int32),
        pltpu.VMEM((128,), jnp.int32),
        pltpu.VMEM((_TSL,), jnp.float32),
    ]
    fn = pl.kernel(_topk_body,
                   out_type=[jax.ShapeDtypeStruct((_NP,), jnp.float32)],
                   mesh=mesh, scratch_types=scratch, name="topk_rank",
                   compiler_params=pltpu.CompilerParams(
                       needs_layout_passes=False))
    return fn(score_flat, starts_row.reshape(128))[0]


def kernel(x, edge_index, batch, Wrel1, brel1, Wroot1, Wrel2, brel2, Wroot2,
           Wrel3, brel3, Wroot3, Wrel4, brel4, Wroot4, pool_attn, lin1_W,
           lin1_b, lin2_W, lin2_b):
    src = edge_index[0].astype(jnp.int32)
    dst = edge_index[1].astype(jnp.int32)
    padn = _NP - _N

    xp = jnp.pad(x, ((0, padn), (0, 0)))
    batp = jnp.pad(batch.astype(jnp.int32), (0, padn),
                   constant_values=_G).reshape(1, _NP)
    ones_nm = jnp.pad(jnp.ones((_N,), jnp.float32), (0, padn)).reshape(1, _NP)

    idx3 = ((2 * src)[None, :]
            + jnp.array([0, 1], jnp.int32)[:, None]).reshape(
                2, _NSUB, _NCH, _CH)
    dump = _HNP + jnp.arange(_E, dtype=jnp.int32) % 8
    in_lo = dst < _HNP
    dst_lo3 = jnp.where(in_lo, dst, dump).reshape(_NSUB, _NCH, _CH)
    dst_hi3 = jnp.where(in_lo, dump, dst - _HNP).reshape(_NSUB, _NCH, _CH)

    pa = pool_attn.reshape(1, _H)
    dstlh = jnp.stack([dst_lo3, dst_hi3])

    wpad = ((0, 0), (0, _H - _F))
    WrS = jnp.stack([jnp.pad(Wrel1, wpad), Wrel2, Wrel3, Wrel4])
    WsS = jnp.stack([jnp.pad(Wroot1, wpad), Wroot2, Wroot3, Wroot4])
    brS = jnp.stack([brel1, brel2, brel3, brel4]).reshape(4, 1, _H)
    poolS = jnp.array([False, True, False, False])
    flagS = jnp.array([0, 1, 1, 1], jnp.int32)

    xp2 = jnp.pad(xp, ((0, 0), (0, _H - _F)))

    ones_bits = jnp.full((1, _BW), -1, jnp.int32)

    def stage(carry, xs):
        h, keepp, kbits = carry
        Wr, Ws, br, is_pool, aflag = xs
        table = h.reshape(2 * _NP, 128)
        bits_in = kbits.at[0, _BW - 1].set(aflag)

        def half(_, dsta):
            return 0, _agg_call(table, idx3, dsta, bits_in)

        _, (aggs, degs) = lax.scan(half, 0, dstlh)
        agg = jnp.concatenate([aggs[0], aggs[1]], axis=1)
        deg = jnp.concatenate([degs[0], degs[1]], axis=2)
        h_next, pool, cnt, score, starts = _conv_call(agg, deg, h, batp,
                                                      keepp, Wr, br, Ws, pa,
                                                      with_score=True)

        def do_pool(args):
            hn, kp0, kb0 = args
            keep = _topk_call(score.reshape(_NP), starts)
            kp = keep.reshape(1, _NP)
            hm = _mask_call(hn, score, kp)
            sh = jnp.arange(32, dtype=jnp.int32)
            words = jnp.sum(
                keep.reshape(_NP // 32, 32).astype(jnp.int32) << sh, axis=1)
            kb = jnp.pad(words, (0, _BW - _NP // 32)).reshape(1, _BW)
            return hm, kp, kb

        h_next, keepp, kbits = lax.cond(is_pool, do_pool, lambda a: a,
                                        (h_next, keepp, kbits))
        return (h_next, keepp, kbits), (pool, cnt)

    (_, _, _), (pools, cnts) = lax.scan(stage, (xp2, ones_nm, ones_bits),
                                        (WrS, WsS, brS, poolS, flagS))

    w2p = jnp.pad(lin2_W, ((0, 128 - _C), (0, 0)))
    b2p = jnp.pad(lin2_b, (0, 128 - _C)).reshape(1, 128)
    out = _head_call(pools[0], pools[1], pools[2], pools[3],
                     cnts[0], cnts[2],
                     lin1_W, lin1_b.reshape(1, _H), w2p, b2p)
    return out[:, :_C]


# speedup vs baseline: 4.6862x; 1.0025x over previous
import functools

import jax
import jax.numpy as jnp
from jax import lax
from jax.experimental import pallas as pl
from jax.experimental.pallas import tpu as pltpu
from jax.experimental.pallas import tpu_sc as plsc

_N = 10000
_E = 320000
_F = 128
_H = 256
_G = 64
_C = 10
_RATIO = 0.8

_NP = 10240
_BLK = 1024
_NB = _NP // _BLK


_NSUB = 16
_EPT = _E // _NSUB
_CH = 80
_NCH = _EPT // _CH
_SLAB = _NP // _NSUB


_HNP = _NP // 2
_ACC = _HNP + 8
_QS = _HNP // _NSUB
_LB = 25
_NRF = _NCH // _LB
_BW = 384


def _agg_body(table_ref, idx_ref, dsta_ref, bits_ref, zeros_ref, zrow_ref,
              agg_out_ref, deg_out_ref, idx_v, dsta_v, rows_a, rows_b,
              bits_v, dacc_v, acc, sem_a, sem_b, sem_c, sem_d):
    c = lax.axis_index("c")
    s = lax.axis_index("s")
    half = _NCH // 2

    pltpu.sync_copy(zeros_ref, acc.at[pl.ds(s * _QS, _QS)])
    pltpu.sync_copy(bits_ref, bits_v)
    pltpu.sync_copy(zrow_ref, dacc_v)
    plsc.subcore_barrier()

    def buf(j2):
        return (rows_a, sem_a) if j2 % 2 == 0 else (rows_b, sem_b)

    def ssem(j2):
        return sem_c if j2 % 2 == 0 else sem_d

    def s_wait(j2):
        bj, _ = buf(j2)
        pltpu.make_async_copy(bj, acc.at[dsta_v.at[j2]], ssem(j2)).wait()

    def refill(r, carry):
        pltpu.sync_copy(idx_ref.at[c, s, r], idx_v)
        pltpu.sync_copy(dsta_ref.at[s, r], dsta_v)
        b0, s0 = buf(0)
        pltpu.async_copy(table_ref.at[idx_v.at[0]], b0, s0)
        for j2 in range(_LB):
            bj, sj = buf(j2)
            pltpu.make_async_copy(table_ref.at[idx_v.at[j2]], bj, sj).wait()
            if j2 + 1 < _LB:
                if j2 >= 1:
                    s_wait(j2 - 1)
                bn, sn = buf(j2 + 1)
                pltpu.async_copy(table_ref.at[idx_v.at[j2 + 1]], bn, sn)
            pltpu.async_copy(bj, acc.at[dsta_v.at[j2]], ssem(j2), add=True)
            j = r * _LB + j2

            @pl.when(jnp.logical_and(j >= c * half, j < (c + 1) * half))
            def _():
                for k in range(_CH // 16):
                    iv = idx_v[j2, pl.ds(k * 16, 16)]
                    sv = jnp.right_shift(iv, 1)
                    dv = dsta_v[j2, pl.ds(k * 16, 16)]
                    w = plsc.load_gather(bits_v, [jnp.right_shift(sv, 5)])
                    kv = jnp.bitwise_and(
                        jnp.right_shift(w, jnp.bitwise_and(sv, 31)),
                        1).astype(jnp.float32)
                    plsc.addupdate_scatter(dacc_v, [dv], kv,
                                           mask=dv < _HNP)

        s_wait(_LB - 2)
        s_wait(_LB - 1)
        return carry

    lax.fori_loop(0, _NRF, refill, 0)

    plsc.subcore_barrier()
    pltpu.sync_copy(acc.at[pl.ds(s * _QS, _QS)],
                    agg_out_ref.at[c, pl.ds(s * _QS, _QS)])
    pltpu.sync_copy(dacc_v, deg_out_ref.at[c, s])


def _agg_call(table, idx3, dsta3, bits):
    zeros = jnp.zeros((_QS, 128), jnp.float32)
    zrow = jnp.zeros((_HNP,), jnp.float32)
    mesh = plsc.VectorSubcoreMesh(core_axis_name="c", subcore_axis_name="s")
    out_type = [jax.ShapeDtypeStruct((2, _HNP, 128), jnp.float32),
                jax.ShapeDtypeStruct((2, _NSUB, _HNP), jnp.float32)]
    scratch = [
        pltpu.VMEM((_LB, _CH), jnp.int32),
        pltpu.VMEM((_LB, _CH), jnp.int32),
        pltpu.VMEM((_CH, 128), jnp.float32),
        pltpu.VMEM((_CH, 128), jnp.float32),
        pltpu.VMEM((_BW,), jnp.int32),
        pltpu.VMEM((_HNP,), jnp.float32),
        pltpu.VMEM_SHARED((_ACC, 128), jnp.float32),
        pltpu.SemaphoreType.DMA,
        pltpu.SemaphoreType.DMA,
        pltpu.SemaphoreType.DMA,
        pltpu.SemaphoreType.DMA,
    ]
    fn = pl.kernel(_agg_body, out_type=out_type, mesh=mesh,
                   scratch_types=scratch, name="edge_agg",
                   compiler_params=pltpu.CompilerParams(
                       needs_layout_passes=False))
    idx5 = idx3.reshape(2, _NSUB, _NRF, _LB, _CH)
    dsta5 = dsta3.reshape(_NSUB, _NRF, _LB, _CH)
    return fn(table, idx5, dsta5, bits.reshape(_BW), zeros, zrow)


def _conv_body(agg_ref, deg_ref, hin_ref, bat_ref, nm_ref, wr_ref, br_ref,
               ws_ref, pa_ref, h_ref, pool_ref, cnt_ref, score_ref,
               starts_ref):
    i = pl.program_id(0)
    d = deg_ref[:, :, pl.ds(i * _BLK, _BLK)]
    deg = jnp.sum(d, axis=(0, 1))
    hin = hin_ref[...]
    nm = nm_ref[0, pl.ds(i * _BLK, _BLK)]
    inv = (1.0 / jnp.maximum(deg, 1.0))[:, None]
    rel = None
    for cc in range(2):
        col = cc * 128
        t = lax.dot_general(agg_ref[cc] * inv, wr_ref[:, col:col + 128],
                            (((1,), (1,)), ((), ())),
                            preferred_element_type=jnp.float32)
        rel = t if rel is None else rel + t
    root = lax.dot_general(hin, ws_ref[...], (((1,), (1,)), ((), ())),
                           preferred_element_type=jnp.float32)
    h = jax.nn.relu(rel + br_ref[0, :][None, :] + root)
    h_ref[...] = h * nm[:, None]

    bat = bat_ref[0, pl.ds(i * _BLK, _BLK)]
    gids = lax.broadcasted_iota(jnp.int32, (_G, _BLK), 0)
    oh = jnp.where(gids == bat[None, :], nm[None, :], 0.0)
    psum = lax.dot_general(oh, h, (((1,), (0,)), ((), ())),
                           preferred_element_type=jnp.float32)
    pcnt = jnp.sum(oh, axis=1)

    @pl.when(i == 0)
    def _():
        pool_ref[...] = jnp.zeros_like(pool_ref)
        cnt_ref[...] = jnp.zeros_like(cnt_ref)

    pool_ref[...] += psum
    row0 = lax.broadcasted_iota(jnp.int32, (8, _G), 0) == 0
    cnt_ref[...] += jnp.where(row0, pcnt[None, :], 0.0)

    pa = pa_ref[0, :]
    nrm = jnp.sqrt(jnp.sum(pa * pa)) + 1e-12
    s = jnp.sum(h * (pa / nrm)[None, :], axis=1)
    score_ref[0, pl.ds(i * _BLK, _BLK)] = s

    @pl.when(i == _NB - 1)
    def _():
        counts = cnt_ref[0, :]
        gp = lax.broadcasted_iota(jnp.int32, (_G, 128), 0)
        ln = lax.broadcasted_iota(jnp.int32, (_G, 128), 1)
        lt = (gp < ln).astype(jnp.float32)
        starts_ref[...] = lax.dot_general(
            counts[None, :], lt, (((1,), (0,)), ((), ())),
            preferred_element_type=jnp.float32)


def _conv_call(agg, deg, hin, bat, nm, Wr, br, Ws, pa, *, with_score=True):
    fin = hin.shape[1]
    out_shape = [
        jax.ShapeDtypeStruct((_NP, _H), jnp.float32),
        jax.ShapeDtypeStruct((_G, _H), jnp.float32),
        jax.ShapeDtypeStruct((8, _G), jnp.float32),
        jax.ShapeDtypeStruct((1, _NP), jnp.float32),
        jax.ShapeDtypeStruct((1, 128), jnp.float32),
    ]
    full = lambda shape: pl.BlockSpec(shape, lambda i: (0, 0))
    return pl.pallas_call(
        _conv_body,
        grid=(_NB,),
        in_specs=[
            pl.BlockSpec((2, _BLK, 128), lambda i: (0, i, 0)),
            pl.BlockSpec((2, _NSUB, _NP), lambda i: (0, 0, 0)),
            pl.BlockSpec((_BLK, fin), lambda i: (i, 0)),
            full((1, _NP)),
            full((1, _NP)),
            full((_H, fin)),
            full((1, _H)),
            full((_H, fin)),
            full((1, _H)),
        ],
        out_specs=[
            pl.BlockSpec((_BLK, _H), lambda i: (i, 0)),
            full((_G, _H)),
            full((8, _G)),
            full((1, _NP)),
            full((1, 128)),
        ],
        out_shape=out_shape,
    )(agg, deg, hin, bat, nm, Wr, br, Ws, pa)


def _mask_body(h_ref, score_ref, keep_ref, out_ref):
    i = pl.program_id(0)
    s = score_ref[0, pl.ds(i * _BLK, _BLK)]
    k = keep_ref[0, pl.ds(i * _BLK, _BLK)]
    out_ref[...] = h_ref[...] * (jnp.tanh(s) * k)[:, None]


def _mask_call(h, score, keep):
    full = lambda shape: pl.BlockSpec(shape, lambda i: (0, 0))
    return pl.pallas_call(
        _mask_body,
        grid=(_NB,),
        in_specs=[
            pl.BlockSpec((_BLK, _H), lambda i: (i, 0)),
            full((1, _NP)),
            full((1, _NP)),
        ],
        out_specs=pl.BlockSpec((_BLK, _H), lambda i: (i, 0)),
        out_shape=jax.ShapeDtypeStruct((_NP, _H), jnp.float32),
    )(h, score, keep)


def _head_body(p1_ref, p2_ref, p3_ref, p4_ref, c0_ref, c1_ref, w1_ref,
               b1_ref, w2_ref, b2_ref, out_ref):
    ic0 = 1.0 / jnp.maximum(c0_ref[0, :], 1.0)
    ic1 = 1.0 / jnp.maximum(c1_ref[0, :], 1.0)
    z = jnp.concatenate([
        p1_ref[...] * ic0[:, None],
        p2_ref[...] * ic0[:, None],
        p3_ref[...] * ic1[:, None],
        p4_ref[...] * ic1[:, None],
    ], axis=1)
    t = lax.dot_general(z, w1_ref[...], (((1,), (1,)), ((), ())),
                        preferred_element_type=jnp.float32)
    t = jax.nn.relu(t + b1_ref[0, :][None, :])
    o = lax.dot_general(t, w2_ref[...], (((1,), (1,)), ((), ())),
                        preferred_element_type=jnp.float32)
    o = o + b2_ref[0, :][None, :]
    lane = lax.broadcasted_iota(jnp.int32, (_G, 128), 1)
    o = jnp.where(lane < _C, o, -jnp.inf)
    m = jnp.max(o, axis=1, keepdims=True)
    e = jnp.where(lane < _C, jnp.exp(o - m), 0.0)
    out_ref[...] = e / jnp.sum(e, axis=1, keepdims=True)


def _head_call(p1, p2, p3, p4, c0, c1, w1, b1, w2, b2):
    full = lambda shape: pl.BlockSpec(shape, lambda: tuple(0 for _ in shape))
    return pl.pallas_call(
        _head_body,
        in_specs=[
            full((_G, _H)), full((_G, _H)), full((_G, _H)), full((_G, _H)),
            full((8, _G)), full((8, _G)),
            full((_H, 4 * _H)), full((1, _H)),
            full((128, _H)), full((1, 128)),
        ],
        out_specs=full((_G, 128)),
        out_shape=jax.ShapeDtypeStruct((_G, 128), jnp.float32),
    )(p1, p2, p3, p4, c0, c1, w1, b1, w2, b2)


_TSL = _NP // 32


def _topk_body(score_ref, starts_in_ref, keep_out_ref, score_v, starts_f,
               starts_v, kcnt_v, keep_v):
    c = lax.axis_index("c")
    s = lax.axis_index("s")
    w = s * 2 + c
    base = w * _TSL
    pltpu.sync_copy(score_ref, score_v)
    pltpu.sync_copy(starts_in_ref, starts_f)

    for k in range(8):
        starts_v[pl.ds(16 * k, 16)] = (
            starts_f[pl.ds(16 * k, 16)].astype(jnp.int32))

    for k in range(8):
        gv = lax.iota(jnp.int32, 16) + 16 * k
        st0 = plsc.load_gather(starts_v, [gv])
        st1 = plsc.load_gather(starts_v, [jnp.minimum(gv + 1, 127)])
        cf = (st1 - st0).astype(jnp.float32)
        v = jnp.float32(_RATIO) * cf
        t = v.astype(jnp.int32)
        kc = t + (t.astype(jnp.float32) < v).astype(jnp.int32)
        kcnt_v[pl.ds(16 * k, 16)] = jnp.where(gv < _G, kc, 0)

    def chunk_body(k, carry):
        b16 = base + k * 16
        i_vec = lax.iota(jnp.int32, 16) + b16
        s_i = score_v[pl.ds(b16, 16)]
        lo = jnp.zeros((16,), jnp.int32)
        hi = jnp.full((16,), 127, jnp.int32)
        for _ in range(7):
            mid = (lo + hi + 1) // 2
            v = plsc.load_gather(starts_v, [mid])
            ok = v <= i_vec
            lo = jnp.where(ok, mid, lo)
            hi = jnp.where(ok, hi, mid - 1)
        b_i = lo
        jlo_v = plsc.load_gather(starts_v, [b_i])
        jhi_v = plsc.load_gather(starts_v, [jnp.minimum(b_i + 1, 127)])
        jmin = lax.reduce_min(jlo_v, (0,))
        jmax = lax.reduce_max(jhi_v, (0,))

        def inner(j, cnt):
            jb = jnp.zeros((16,), jnp.int32) + j
            sj = plsc.load_gather(score_v, [jb])
            gt = jnp.logical_or(sj > s_i,
                                jnp.logical_and(sj == s_i, j < i_vec))
            m = jnp.logical_and(gt, jnp.logical_and(j >= jlo_v, j < jhi_v))
            return cnt + m.astype(jnp.int32)

        cnt = lax.fori_loop(jmin, jmax, inner, jnp.zeros((16,), jnp.int32))
        kc = plsc.load_gather(kcnt_v, [b_i])
        keep_v[pl.ds(k * 16, 16)] = (cnt < kc).astype(jnp.float32)
        return carry

    lax.fori_loop(0, _TSL // 16, chunk_body, 0)
    pltpu.sync_copy(keep_v, keep_out_ref.at[pl.ds(base, _TSL)])


def _topk_call(score_flat, starts_row):
    mesh = plsc.VectorSubcoreMesh(core_axis_name="c", subcore_axis_name="s")
    scratch = [
        pltpu.VMEM((_NP,), jnp.float32),
        pltpu.VMEM((128,), jnp.float32),
        pltpu.VMEM((128,), jnp.int32),
        pltpu.VMEM((128,), jnp.int32),
        pltpu.VMEM((_TSL,), jnp.float32),
    ]
    fn = pl.kernel(_topk_body,
                   out_type=[jax.ShapeDtypeStruct((_NP,), jnp.float32)],
                   mesh=mesh, scratch_types=scratch, name="topk_rank",
                   compiler_params=pltpu.CompilerParams(
                       needs_layout_passes=False))
    return fn(score_flat, starts_row.reshape(128))[0]


def kernel(x, edge_index, batch, Wrel1, brel1, Wroot1, Wrel2, brel2, Wroot2,
           Wrel3, brel3, Wroot3, Wrel4, brel4, Wroot4, pool_attn, lin1_W,
           lin1_b, lin2_W, lin2_b):
    src = edge_index[0].astype(jnp.int32)
    dst = edge_index[1].astype(jnp.int32)
    padn = _NP - _N

    xp = jnp.pad(x, ((0, padn), (0, 0)))
    batp = jnp.pad(batch.astype(jnp.int32), (0, padn),
                   constant_values=_G).reshape(1, _NP)
    ones_nm = jnp.pad(jnp.ones((_N,), jnp.float32), (0, padn)).reshape(1, _NP)

    idx3 = ((2 * src)[None, :]
            + jnp.array([0, 1], jnp.int32)[:, None]).reshape(
                2, _NSUB, _NCH, _CH)
    dump = _HNP + jnp.arange(_E, dtype=jnp.int32) % 8
    in_lo = dst < _HNP
    dst_lo3 = jnp.where(in_lo, dst, dump).reshape(_NSUB, _NCH, _CH)
    dst_hi3 = jnp.where(in_lo, dump, dst - _HNP).reshape(_NSUB, _NCH, _CH)

    pa = pool_attn.reshape(1, _H)
    dstlh = jnp.stack([dst_lo3, dst_hi3])

    wpad = ((0, 0), (0, _H - _F))
    WrS = jnp.stack([jnp.pad(Wrel1, wpad), Wrel2, Wrel3, Wrel4])
    WsS = jnp.stack([jnp.pad(Wroot1, wpad), Wroot2, Wroot3, Wroot4])
    brS = jnp.stack([brel1, brel2, brel3, brel4]).reshape(4, 1, _H)
    poolS = jnp.array([False, True, False, False])

    xp2 = jnp.pad(xp, ((0, 0), (0, _H - _F)))

    ones_bits = jnp.full((1, _BW), -1, jnp.int32)

    def stage(carry, xs):
        h, keepp, kbits = carry
        Wr, Ws, br, is_pool = xs
        table = h.reshape(2 * _NP, 128)

        def half(_, dsta):
            return 0, _agg_call(table, idx3, dsta, kbits)

        _, (aggs, degs) = lax.scan(half, 0, dstlh)
        agg = jnp.concatenate([aggs[0], aggs[1]], axis=1)
        deg = jnp.concatenate([degs[0], degs[1]], axis=2)
        h_next, pool, cnt, score, starts = _conv_call(agg, deg, h, batp,
                                                      keepp, Wr, br, Ws, pa,
                                                      with_score=True)

        def do_pool(args):
            hn, kp0, kb0 = args
            keep = _topk_call(score.reshape(_NP), starts)
            kp = keep.reshape(1, _NP)
            hm = _mask_call(hn, score, kp)
            sh = jnp.arange(32, dtype=jnp.int32)
            words = jnp.sum(
                keep.reshape(_NP // 32, 32).astype(jnp.int32) << sh, axis=1)
            kb = jnp.pad(words, (0, _BW - _NP // 32)).reshape(1, _BW)
            return hm, kp, kb

        h_next, keepp, kbits = lax.cond(is_pool, do_pool, lambda a: a,
                                        (h_next, keepp, kbits))
        return (h_next, keepp, kbits), (pool, cnt)

    (_, _, _), (pools, cnts) = lax.scan(stage, (xp2, ones_nm, ones_bits),
                                        (WrS, WsS, brS, poolS))

    w2p = jnp.pad(lin2_W, ((0, 128 - _C), (0, 0)))
    b2p = jnp.pad(lin2_b, (0, 128 - _C)).reshape(1, 128)
    out = _head_call(pools[0], pools[1], pools[2], pools[3],
                     cnts[0], cnts[2],
                     lin1_W, lin1_b.reshape(1, _H), w2p, b2p)
    return out[:, :_C]
